# baseline (speedup 1.0000x reference)
import functools

import jax
import jax.numpy as jnp
from jax import lax
from jax.experimental import pallas as pl
from jax.experimental.pallas import tpu as pltpu
from jax.experimental.pallas import tpu_sc as plsc

N = 8192
D = 1024
E = 8
K = 2
LOSS_COEF = 0.01

PAIRS = N * K
TILE = 256
P_BUF = PAIRS + E * TILE
NT = P_BUF // TILE
NW = 32
CHUNK_P = PAIRS // NW
ROWS_W = P_BUF // NW
GCH = 64
TOK_W = N // NW
CCH = 32

_GATE_TILE = 1024

@functools.cache
def _sc_mesh():
    return plsc.VectorSubcoreMesh(
        core_axis_name="c", subcore_axis_name="s",
        num_cores=2, num_subcores=16)


def _gating_kernel(x_ref, wg_ref, b_ref, gates_ref, psum_ref,
                   i1_ref, i2_ref, g1_ref, g2_ref):
    logits = jnp.dot(x_ref[...], wg_ref[...],
                     preferred_element_type=jnp.float32) + b_ref[...]
    iota = lax.broadcasted_iota(jnp.int32, logits.shape, 1)
    v1 = jnp.max(logits, axis=1, keepdims=True)
    i1 = jnp.argmax(logits, axis=1).astype(jnp.int32)[:, None]
    masked = jnp.where(iota == i1, -jnp.inf, logits)
    v2 = jnp.max(masked, axis=1, keepdims=True)
    i2 = jnp.argmax(masked, axis=1).astype(jnp.int32)[:, None]
    e2 = jnp.exp(v2 - v1)
    g1 = 1.0 / (1.0 + e2)
    g2 = e2 * g1
    gates = jnp.where(iota == i1, g1, 0.0) + jnp.where(iota == i2, g2, 0.0)
    gates_ref[...] = gates
    psum_ref[0, 0, :] = jnp.sum(gates, axis=0)
    i1_ref[...] = i1
    i2_ref[...] = i2
    g1_ref[...] = g1
    g2_ref[...] = g2


def _routing_kernel(ia_ref, ib_ref, dest_ref, teid_ref):
    ids = jnp.concatenate([ia_ref[...], ib_ref[...]], axis=0)
    r_i = lax.broadcasted_iota(jnp.int32, (128, 128), 0)
    c_i = lax.broadcasted_iota(jnp.int32, (128, 128), 1)
    triu = (r_i <= c_i).astype(jnp.float32)
    low = (r_i > c_i).astype(jnp.float32)
    masks = [ids == e for e in range(E)]
    counts = [jnp.sum(m.astype(jnp.float32)).astype(jnp.int32) for m in masks]
    pcs = [((c + TILE - 1) // TILE) * TILE for c in counts]
    offs = [jnp.int32(0)]
    for e in range(E - 1):
        offs.append(offs[-1] + pcs[e])
    dest = jnp.zeros((128, 128), jnp.float32)
    for e in range(E):
        mf = masks[e].astype(jnp.float32)
        row_incl = jnp.dot(mf, triu, preferred_element_type=jnp.float32)
        prev = jnp.dot(low, row_incl[:, 127:128],
                       preferred_element_type=jnp.float32)
        rank_incl = row_incl + prev
        dest = jnp.where(masks[e],
                         offs[e].astype(jnp.float32) + rank_incl - 1.0, dest)
    dest_ref[...] = dest.astype(jnp.int32)
    t_iota = lax.broadcasted_iota(jnp.int32, (1, NT), 1) * TILE
    acc = jnp.zeros((1, NT), jnp.int32)
    cum = jnp.int32(0)
    for e in range(E):
        cum = cum + pcs[e]
        acc = acc + (t_iota >= cum).astype(jnp.int32)
    teid_ref[...] = jnp.minimum(acc, E - 1)


def _scatter_body(dest_hbm, gw_hbm, perm_hbm, gws_hbm,
                  didx_v, tok_v, gw_v, sem1, sem2):
    w = lax.axis_index("s") * 2 + lax.axis_index("c")
    base = w * CHUNK_P
    pltpu.sync_copy(dest_hbm.at[pl.ds(base, CHUNK_P)], didx_v)
    pltpu.sync_copy(gw_hbm.at[pl.ds(base, CHUNK_P)], gw_v)
    tokbase = base - jnp.where(w >= NW // 2, PAIRS // 2, 0)
    i16 = lax.broadcasted_iota(jnp.int32, (16,), 0)
    for j in range(CHUNK_P // 16):
        tok_v[pl.ds(j * 16, 16)] = i16 + (tokbase + j * 16)
    cp1 = pltpu.make_async_copy(tok_v, perm_hbm.at[didx_v], sem1)
    cp1.start()
    cp2 = pltpu.make_async_copy(gw_v, gws_hbm.at[didx_v], sem2)
    cp2.start()
    cp1.wait()
    cp2.wait()


def _gather_body(perm_hbm, x_hbm, xs_hbm, idx_v, rows_v, sem):
    w = lax.axis_index("s") * 2 + lax.axis_index("c")
    base = w * ROWS_W
    for cid in range(ROWS_W // GCH):
        pltpu.sync_copy(perm_hbm.at[pl.ds(base + cid * GCH, GCH)], idx_v)
        for j in range(GCH // 16):
            v = idx_v[pl.ds(j * 16, 16)]
            idx_v[pl.ds(j * 16, 16)] = jnp.minimum(
                jnp.maximum(v, 0), N - 1)
        pltpu.async_copy(x_hbm.at[idx_v], rows_v, sem).wait()
        pltpu.sync_copy(rows_v, xs_hbm.at[pl.ds(base + cid * GCH, GCH)])


def _mm_kernel(eid_ref, xs_ref, w_ref, b_ref, g_ref, o_ref):
    del eid_ref
    o_ref[...] = (jnp.dot(xs_ref[...], w_ref[0],
                          preferred_element_type=jnp.float32)
                  + b_ref[0]) * g_ref[0]


def _combine_body(da_hbm, db_hbm, ys_hbm, y_hbm,
                  ia_v, ib_v, ra_v, rb_v, sema, semb):
    w = lax.axis_index("s") * 2 + lax.axis_index("c")
    tb = w * TOK_W
    for cid in range(TOK_W // CCH):
        pltpu.sync_copy(da_hbm.at[pl.ds(tb + cid * CCH, CCH)], ia_v)
        pltpu.sync_copy(db_hbm.at[pl.ds(tb + cid * CCH, CCH)], ib_v)
        ca = pltpu.make_async_copy(ys_hbm.at[ia_v], ra_v, sema)
        ca.start()
        cb = pltpu.make_async_copy(ys_hbm.at[ib_v], rb_v, semb)
        cb.start()
        ca.wait()
        cb.wait()

        def addrow(r, carry):
            for j in range(D // 16):
                ra_v[r, pl.ds(j * 16, 16)] = (
                    ra_v[r, pl.ds(j * 16, 16)] + rb_v[r, pl.ds(j * 16, 16)])
            return carry

        lax.fori_loop(0, CCH, addrow, 0)
        pltpu.sync_copy(ra_v, y_hbm.at[pl.ds(tb + cid * CCH, CCH)])


@jax.jit
def kernel(x, w_gate_W, w_gate_b, expert_W, expert_b):
    n_gt = N // _GATE_TILE
    gates, psums, i1, i2, g1, g2 = pl.pallas_call(
        _gating_kernel,
        grid=(n_gt,),
        in_specs=[
            pl.BlockSpec((_GATE_TILE, D), lambda i: (i, 0)),
            pl.BlockSpec((D, E), lambda i: (0, 0)),
            pl.BlockSpec((1, E), lambda i: (0, 0)),
        ],
        out_specs=[
            pl.BlockSpec((_GATE_TILE, E), lambda i: (i, 0)),
            pl.BlockSpec((1, 1, E), lambda i: (i, 0, 0)),
            pl.BlockSpec((_GATE_TILE, 1), lambda i: (i, 0)),
            pl.BlockSpec((_GATE_TILE, 1), lambda i: (i, 0)),
            pl.BlockSpec((_GATE_TILE, 1), lambda i: (i, 0)),
            pl.BlockSpec((_GATE_TILE, 1), lambda i: (i, 0)),
        ],
        out_shape=[
            jax.ShapeDtypeStruct((N, E), jnp.float32),
            jax.ShapeDtypeStruct((n_gt, 1, E), jnp.float32),
            jax.ShapeDtypeStruct((N, 1), jnp.int32),
            jax.ShapeDtypeStruct((N, 1), jnp.int32),
            jax.ShapeDtypeStruct((N, 1), jnp.float32),
            jax.ShapeDtypeStruct((N, 1), jnp.float32),
        ],
        compiler_params=pltpu.CompilerParams(
            dimension_semantics=("parallel",)),
    )(x, w_gate_W, w_gate_b.reshape(1, E))

    dest128, teid = pl.pallas_call(
        _routing_kernel,
        grid=(1,),
        in_specs=[
            pl.BlockSpec((64, 128), lambda i: (0, 0)),
            pl.BlockSpec((64, 128), lambda i: (0, 0)),
        ],
        out_specs=[
            pl.BlockSpec((128, 128), lambda i: (0, 0)),
            pl.BlockSpec((1, NT), lambda i: (0, 0)),
        ],
        out_shape=[
            jax.ShapeDtypeStruct((128, 128), jnp.int32),
            jax.ShapeDtypeStruct((1, NT), jnp.int32),
        ],
    )(i1.reshape(64, 128), i2.reshape(64, 128))

    dest_flat = dest128.reshape(PAIRS)
    gw_flat = jnp.concatenate([g1.reshape(N), g2.reshape(N)])

    perm, gws = pl.kernel(
        _scatter_body,
        out_type=[
            jax.ShapeDtypeStruct((P_BUF,), jnp.int32),
            jax.ShapeDtypeStruct((P_BUF,), jnp.float32),
        ],
        mesh=_sc_mesh(),
        scratch_types=[
            pltpu.VMEM((CHUNK_P,), jnp.int32),
            pltpu.VMEM((CHUNK_P,), jnp.int32),
            pltpu.VMEM((CHUNK_P,), jnp.float32),
            pltpu.SemaphoreType.DMA,
            pltpu.SemaphoreType.DMA,
        ],
    )(dest_flat, gw_flat)

    xs = pl.kernel(
        _gather_body,
        out_type=jax.ShapeDtypeStruct((P_BUF, D), jnp.float32),
        mesh=_sc_mesh(),
        scratch_types=[
            pltpu.VMEM((GCH,), jnp.int32),
            pltpu.VMEM((GCH, D), jnp.float32),
            pltpu.SemaphoreType.DMA,
        ],
    )(perm, x)

    ys = pl.pallas_call(
        _mm_kernel,
        grid_spec=pltpu.PrefetchScalarGridSpec(
            num_scalar_prefetch=1,
            grid=(NT,),
            in_specs=[
                pl.BlockSpec((TILE, D), lambda i, eid: (i, 0)),
                pl.BlockSpec((1, D, D), lambda i, eid: (eid[i], 0, 0)),
                pl.BlockSpec((1, 1, D), lambda i, eid: (eid[i], 0, 0)),
                pl.BlockSpec((1, TILE, 1), lambda i, eid: (i, 0, 0)),
            ],
            out_specs=pl.BlockSpec((TILE, D), lambda i, eid: (i, 0)),
        ),
        out_shape=jax.ShapeDtypeStruct((P_BUF, D), jnp.float32),
        compiler_params=pltpu.CompilerParams(
            dimension_semantics=("arbitrary",)),
    )(teid.reshape(NT), xs, expert_W, expert_b.reshape(E, 1, D),
      gws.reshape(NT, TILE, 1))

    y = pl.kernel(
        _combine_body,
        out_type=jax.ShapeDtypeStruct((N, D), jnp.float32),
        mesh=_sc_mesh(),
        scratch_types=[
            pltpu.VMEM((CCH,), jnp.int32),
            pltpu.VMEM((CCH,), jnp.int32),
            pltpu.VMEM((CCH, D), jnp.float32),
            pltpu.VMEM((CCH, D), jnp.float32),
            pltpu.SemaphoreType.DMA,
            pltpu.SemaphoreType.DMA,
        ],
    )(dest_flat[:N], dest_flat[N:], ys)

    importance = jnp.sum(psums[:, 0, :], axis=0) / N
    loss = (jnp.std(importance, ddof=1) / jnp.mean(importance)) * LOSS_COEF
    return (y, loss, gates)


# speedup vs baseline: 1.2472x; 1.2472x over previous
import functools

import jax
import jax.numpy as jnp
from jax import lax
from jax.experimental import pallas as pl
from jax.experimental.pallas import tpu as pltpu
from jax.experimental.pallas import tpu_sc as plsc

N = 8192
D = 1024
E = 8
K = 2
LOSS_COEF = 0.01

PAIRS = N * K
TILE = 256
P_BUF = PAIRS + E * TILE
NT = P_BUF // TILE
NW = 32
ROWS_W = P_BUF // NW
GCH = 48
TOK_W = N // NW
CCH = 16

_GATE_TILE = 1024

@functools.cache
def _sc_mesh():
    return plsc.VectorSubcoreMesh(
        core_axis_name="c", subcore_axis_name="s",
        num_cores=2, num_subcores=16)


def _gating_kernel(x_ref, wg_ref, b_ref, gates_ref, psum_ref,
                   i1_ref, i2_ref, g1_ref, g2_ref):
    logits = jnp.dot(x_ref[...], wg_ref[...],
                     preferred_element_type=jnp.float32) + b_ref[...]
    iota = lax.broadcasted_iota(jnp.int32, logits.shape, 1)
    v1 = jnp.max(logits, axis=1, keepdims=True)
    i1 = jnp.argmax(logits, axis=1).astype(jnp.int32)[:, None]
    masked = jnp.where(iota == i1, -jnp.inf, logits)
    v2 = jnp.max(masked, axis=1, keepdims=True)
    i2 = jnp.argmax(masked, axis=1).astype(jnp.int32)[:, None]
    e2 = jnp.exp(v2 - v1)
    g1 = 1.0 / (1.0 + e2)
    g2 = e2 * g1
    gates = jnp.where(iota == i1, g1, 0.0) + jnp.where(iota == i2, g2, 0.0)
    gates_ref[...] = gates
    psum_ref[0, 0, :] = jnp.sum(gates, axis=0)
    i1_ref[...] = i1
    i2_ref[...] = i2
    g1_ref[...] = g1
    g2_ref[...] = g2


def _routing_kernel(ia_ref, ib_ref, dest_ref, teid_ref):
    ids = jnp.concatenate([ia_ref[...], ib_ref[...]], axis=0)
    r_i = lax.broadcasted_iota(jnp.int32, (128, 128), 0)
    c_i = lax.broadcasted_iota(jnp.int32, (128, 128), 1)
    triu = (r_i <= c_i).astype(jnp.float32)
    low = (r_i > c_i).astype(jnp.float32)
    masks = [ids == e for e in range(E)]
    counts = [jnp.sum(m.astype(jnp.float32)).astype(jnp.int32) for m in masks]
    pcs = [((c + TILE - 1) // TILE) * TILE for c in counts]
    offs = [jnp.int32(0)]
    for e in range(E - 1):
        offs.append(offs[-1] + pcs[e])
    dest = jnp.zeros((128, 128), jnp.float32)
    for e in range(E):
        mf = masks[e].astype(jnp.float32)
        row_incl = jnp.dot(mf, triu, preferred_element_type=jnp.float32)
        prev = jnp.dot(low, row_incl[:, 127:128],
                       preferred_element_type=jnp.float32)
        rank_incl = row_incl + prev
        dest = jnp.where(masks[e],
                         offs[e].astype(jnp.float32) + rank_incl - 1.0, dest)
    dest_ref[...] = dest.astype(jnp.int32)
    t_iota = lax.broadcasted_iota(jnp.int32, (1, NT), 1) * TILE
    acc = jnp.zeros((1, NT), jnp.int32)
    cum = jnp.int32(0)
    for e in range(E):
        cum = cum + pcs[e]
        acc = acc + (t_iota >= cum).astype(jnp.int32)
    teid_ref[...] = jnp.minimum(acc, E - 1)


def _dispatch_body(dest_hbm, gw_hbm, x_hbm, xs_hbm, gws_hbm,
                   sh_perm, sh_gws, didx_v, tok_v, gw_v, idx_v,
                   buf0, buf1, sem_s1, sem_s2, sem_g0, sem_g1):
    c = lax.axis_index("c")
    s = lax.axis_index("s")
    w = s * 2 + c
    psl = PAIRS // 16
    pbase = s * psl
    pltpu.sync_copy(dest_hbm.at[pl.ds(pbase, psl)], didx_v)
    pltpu.sync_copy(gw_hbm.at[pl.ds(pbase, psl)], gw_v)
    tokbase = pbase - jnp.where(s >= 8, PAIRS // 2, 0)
    i16 = lax.broadcasted_iota(jnp.int32, (16,), 0)
    for j in range(psl // 16):
        tok_v[pl.ds(j * 16, 16)] = i16 + (tokbase + j * 16)
    cp1 = pltpu.make_async_copy(tok_v, sh_perm.at[didx_v], sem_s1)
    cp1.start()
    cp2 = pltpu.make_async_copy(gw_v, sh_gws.at[didx_v], sem_s2)
    cp2.start()
    cp1.wait()
    cp2.wait()
    plsc.subcore_barrier()
    osl = P_BUF // 16

    @pl.when(c == 0)
    def _():
        pltpu.sync_copy(sh_gws.at[pl.ds(s * osl, osl)],
                        gws_hbm.at[pl.ds(s * osl, osl)])

    base = w * ROWS_W
    pltpu.sync_copy(sh_perm.at[pl.ds(base, ROWS_W)], idx_v)
    for j in range(ROWS_W // 16):
        v = idx_v[pl.ds(j * 16, 16)]
        idx_v[pl.ds(j * 16, 16)] = jnp.minimum(jnp.maximum(v, 0), N - 1)
    bufs = (buf0, buf1)
    sems = (sem_g0, sem_g1)
    cps = [None, None]

    def _fire(cid):
        b = cid % 2
        cps[b] = pltpu.make_async_copy(
            x_hbm.at[idx_v.at[pl.ds(cid * GCH, GCH)]], bufs[b], sems[b])
        cps[b].start()

    _fire(0)
    for cid in range(ROWS_W // GCH):
        b = cid % 2
        cps[b].wait()
        if cid + 1 < ROWS_W // GCH:
            _fire(cid + 1)
        pltpu.sync_copy(bufs[b], xs_hbm.at[pl.ds(base + cid * GCH, GCH)])


def _mm_kernel(eid_ref, xs_ref, w_ref, b_ref, g_ref, o_ref):
    del eid_ref
    o_ref[...] = (jnp.dot(xs_ref[...], w_ref[0],
                          preferred_element_type=jnp.float32)
                  + b_ref[0]) * g_ref[0]


def _combine_body(da_hbm, db_hbm, ys_hbm, y_hbm,
                  idx_all, buf0, buf1, sem0, sem1):
    w = lax.axis_index("s") * 2 + lax.axis_index("c")
    tb = w * TOK_W
    ncc = TOK_W // CCH
    pltpu.sync_copy(da_hbm.at[pl.ds(tb, TOK_W)], idx_all.at[pl.ds(0, TOK_W)])
    pltpu.sync_copy(db_hbm.at[pl.ds(tb, TOK_W)],
                    idx_all.at[pl.ds(TOK_W, TOK_W)])
    for cid in range(ncc):
        for j in range(CCH // 16):
            src_a = idx_all[pl.ds(cid * CCH + j * 16, 16)]
            src_b = idx_all[pl.ds(TOK_W + cid * CCH + j * 16, 16)]
            idx_all[pl.ds(2 * TOK_W + cid * 2 * CCH + j * 16, 16)] = src_a
            idx_all[pl.ds(2 * TOK_W + cid * 2 * CCH + CCH + j * 16, 16)] = (
                src_b)
    bufs = (buf0, buf1)
    sems = (sem0, sem1)
    cps = [None, None]

    def _fire(cid):
        b = cid % 2
        cps[b] = pltpu.make_async_copy(
            ys_hbm.at[idx_all.at[pl.ds(2 * TOK_W + cid * 2 * CCH, 2 * CCH)]],
            bufs[b], sems[b])
        cps[b].start()

    _fire(0)
    for cid in range(ncc):
        b = cid % 2
        cps[b].wait()
        if cid + 1 < ncc:
            _fire(cid + 1)

        def addrow(r, carry):
            for j in range(D // 16):
                bufs[b][r, pl.ds(j * 16, 16)] = (
                    bufs[b][r, pl.ds(j * 16, 16)]
                    + bufs[b][r + CCH, pl.ds(j * 16, 16)])
            return carry

        lax.fori_loop(0, CCH, addrow, 0)
        pltpu.sync_copy(bufs[b].at[pl.ds(0, CCH)],
                        y_hbm.at[pl.ds(tb + cid * CCH, CCH)])


@jax.jit
def kernel(x, w_gate_W, w_gate_b, expert_W, expert_b):
    n_gt = N // _GATE_TILE
    gates, psums, i1, i2, g1, g2 = pl.pallas_call(
        _gating_kernel,
        grid=(n_gt,),
        in_specs=[
            pl.BlockSpec((_GATE_TILE, D), lambda i: (i, 0)),
            pl.BlockSpec((D, E), lambda i: (0, 0)),
            pl.BlockSpec((1, E), lambda i: (0, 0)),
        ],
        out_specs=[
            pl.BlockSpec((_GATE_TILE, E), lambda i: (i, 0)),
            pl.BlockSpec((1, 1, E), lambda i: (i, 0, 0)),
            pl.BlockSpec((_GATE_TILE, 1), lambda i: (i, 0)),
            pl.BlockSpec((_GATE_TILE, 1), lambda i: (i, 0)),
            pl.BlockSpec((_GATE_TILE, 1), lambda i: (i, 0)),
            pl.BlockSpec((_GATE_TILE, 1), lambda i: (i, 0)),
        ],
        out_shape=[
            jax.ShapeDtypeStruct((N, E), jnp.float32),
            jax.ShapeDtypeStruct((n_gt, 1, E), jnp.float32),
            jax.ShapeDtypeStruct((N, 1), jnp.int32),
            jax.ShapeDtypeStruct((N, 1), jnp.int32),
            jax.ShapeDtypeStruct((N, 1), jnp.float32),
            jax.ShapeDtypeStruct((N, 1), jnp.float32),
        ],
        compiler_params=pltpu.CompilerParams(
            dimension_semantics=("parallel",)),
    )(x, w_gate_W, w_gate_b.reshape(1, E))

    dest128, teid = pl.pallas_call(
        _routing_kernel,
        grid=(1,),
        in_specs=[
            pl.BlockSpec((64, 128), lambda i: (0, 0)),
            pl.BlockSpec((64, 128), lambda i: (0, 0)),
        ],
        out_specs=[
            pl.BlockSpec((128, 128), lambda i: (0, 0)),
            pl.BlockSpec((1, NT), lambda i: (0, 0)),
        ],
        out_shape=[
            jax.ShapeDtypeStruct((128, 128), jnp.int32),
            jax.ShapeDtypeStruct((1, NT), jnp.int32),
        ],
    )(i1.reshape(64, 128), i2.reshape(64, 128))

    dest_flat = dest128.reshape(PAIRS)
    gw_flat = jnp.concatenate([g1.reshape(N), g2.reshape(N)])

    psl = PAIRS // 16
    xs, gws = pl.kernel(
        _dispatch_body,
        out_type=[
            jax.ShapeDtypeStruct((P_BUF, D), jnp.float32),
            jax.ShapeDtypeStruct((P_BUF,), jnp.float32),
        ],
        mesh=_sc_mesh(),
        scratch_types=[
            pltpu.VMEM_SHARED((P_BUF,), jnp.int32),
            pltpu.VMEM_SHARED((P_BUF,), jnp.float32),
            pltpu.VMEM((psl,), jnp.int32),
            pltpu.VMEM((psl,), jnp.int32),
            pltpu.VMEM((psl,), jnp.float32),
            pltpu.VMEM((ROWS_W,), jnp.int32),
            pltpu.VMEM((GCH, D), jnp.float32),
            pltpu.VMEM((GCH, D), jnp.float32),
            pltpu.SemaphoreType.DMA,
            pltpu.SemaphoreType.DMA,
            pltpu.SemaphoreType.DMA,
            pltpu.SemaphoreType.DMA,
        ],
    )(dest_flat, gw_flat, x)

    ys = pl.pallas_call(
        _mm_kernel,
        grid_spec=pltpu.PrefetchScalarGridSpec(
            num_scalar_prefetch=1,
            grid=(NT,),
            in_specs=[
                pl.BlockSpec((TILE, D), lambda i, eid: (i, 0)),
                pl.BlockSpec((1, D, D), lambda i, eid: (eid[i], 0, 0)),
                pl.BlockSpec((1, 1, D), lambda i, eid: (eid[i], 0, 0)),
                pl.BlockSpec((1, TILE, 1), lambda i, eid: (i, 0, 0)),
            ],
            out_specs=pl.BlockSpec((TILE, D), lambda i, eid: (i, 0)),
        ),
        out_shape=jax.ShapeDtypeStruct((P_BUF, D), jnp.float32),
        compiler_params=pltpu.CompilerParams(
            dimension_semantics=("arbitrary",)),
    )(teid.reshape(NT), xs, expert_W, expert_b.reshape(E, 1, D),
      gws.reshape(NT, TILE, 1))

    y = pl.kernel(
        _combine_body,
        out_type=jax.ShapeDtypeStruct((N, D), jnp.float32),
        mesh=_sc_mesh(),
        scratch_types=[
            pltpu.VMEM((4 * TOK_W,), jnp.int32),
            pltpu.VMEM((2 * CCH, D), jnp.float32),
            pltpu.VMEM((2 * CCH, D), jnp.float32),
            pltpu.SemaphoreType.DMA,
            pltpu.SemaphoreType.DMA,
        ],
    )(dest_flat[:N], dest_flat[N:], ys)

    importance = jnp.sum(psums[:, 0, :], axis=0) / N
    loss = (jnp.std(importance, ddof=1) / jnp.mean(importance)) * LOSS_COEF
    return (y, loss, gates)


# speedup vs baseline: 2.2843x; 1.8316x over previous
import jax
import jax.numpy as jnp
from jax.experimental import pallas as pl
from jax.experimental.pallas import tpu as pltpu

N = 8192
D = 1024
E = 8
K = 2
LOSS_COEF = 0.01

_GATE_TILE = 1024
_COMB_TILE = 1024


def _gating_kernel(x_ref, wg_ref, b_ref, gates_ref, psum_ref):
    logits = jnp.dot(x_ref[...], wg_ref[...],
                     preferred_element_type=jnp.float32) + b_ref[...]
    iota = jax.lax.broadcasted_iota(jnp.int32, logits.shape, 1)
    v1 = jnp.max(logits, axis=1, keepdims=True)
    i1 = jnp.argmax(logits, axis=1).astype(jnp.int32)[:, None]
    masked = jnp.where(iota == i1, -jnp.inf, logits)
    v2 = jnp.max(masked, axis=1, keepdims=True)
    i2 = jnp.argmax(masked, axis=1).astype(jnp.int32)[:, None]
    e2 = jnp.exp(v2 - v1)
    g1 = 1.0 / (1.0 + e2)
    g2 = e2 * g1
    gates = jnp.where(iota == i1, g1, 0.0) + jnp.where(iota == i2, g2, 0.0)
    gates_ref[...] = gates
    psum_ref[0, 0, :] = jnp.sum(gates, axis=0)


def _combine_kernel(x_ref, g_ref, w_ref, b_ref, o_ref):
    e = pl.program_id(1)

    @pl.when(e == 0)
    def _():
        o_ref[...] = jnp.zeros_like(o_ref)

    xw = jnp.dot(x_ref[...], w_ref[e],
                 preferred_element_type=jnp.float32) + b_ref[0]
    iota = jax.lax.broadcasted_iota(jnp.int32, g_ref.shape, 1)
    gcol = jnp.sum(jnp.where(iota == e, g_ref[...], 0.0), axis=1,
                   keepdims=True)
    o_ref[...] += gcol * xw


@jax.jit
def kernel(x, w_gate_W, w_gate_b, expert_W, expert_b):
    n_gt = N // _GATE_TILE
    gates, psums = pl.pallas_call(
        _gating_kernel,
        grid=(n_gt,),
        in_specs=[
            pl.BlockSpec((_GATE_TILE, D), lambda i: (i, 0)),
            pl.BlockSpec((D, E), lambda i: (0, 0)),
            pl.BlockSpec((1, E), lambda i: (0, 0)),
        ],
        out_specs=[
            pl.BlockSpec((_GATE_TILE, E), lambda i: (i, 0)),
            pl.BlockSpec((1, 1, E), lambda i: (i, 0, 0)),
        ],
        out_shape=[
            jax.ShapeDtypeStruct((N, E), jnp.float32),
            jax.ShapeDtypeStruct((n_gt, 1, E), jnp.float32),
        ],
        compiler_params=pltpu.CompilerParams(
            dimension_semantics=("parallel",)),
    )(x, w_gate_W, w_gate_b.reshape(1, E))

    n_ct = N // _COMB_TILE
    y = pl.pallas_call(
        _combine_kernel,
        grid=(n_ct, E),
        in_specs=[
            pl.BlockSpec((_COMB_TILE, D), lambda i, e: (i, 0)),
            pl.BlockSpec((_COMB_TILE, E), lambda i, e: (i, 0)),
            pl.BlockSpec((E, D, D), lambda i, e: (0, 0, 0)),
            pl.BlockSpec((1, 1, D), lambda i, e: (e, 0, 0)),
        ],
        out_specs=pl.BlockSpec((_COMB_TILE, D), lambda i, e: (i, 0)),
        out_shape=jax.ShapeDtypeStruct((N, D), jnp.float32),
        compiler_params=pltpu.CompilerParams(
            dimension_semantics=("parallel", "arbitrary")),
    )(x, gates, expert_W, expert_b.reshape(E, 1, D))

    importance = jnp.sum(psums[:, 0, :], axis=0) / N
    loss = (jnp.std(importance, ddof=1) / jnp.mean(importance)) * LOSS_COEF
    return (y, loss, gates)


# speedup vs baseline: 2.4297x; 1.0637x over previous
import jax
import jax.numpy as jnp
from jax.experimental import pallas as pl
from jax.experimental.pallas import tpu as pltpu

N = 8192
D = 1024
E = 8
K = 2
LOSS_COEF = 0.01

_GATE_TILE = 1024
_COMB_TILE = 2048


def _gating_kernel(x_ref, wg_ref, b_ref, gates_ref, psum_ref):
    logits = jnp.dot(x_ref[...], wg_ref[...],
                     preferred_element_type=jnp.float32) + b_ref[...]
    iota = jax.lax.broadcasted_iota(jnp.int32, logits.shape, 1)
    v1 = jnp.max(logits, axis=1, keepdims=True)
    i1 = jnp.argmax(logits, axis=1).astype(jnp.int32)[:, None]
    masked = jnp.where(iota == i1, -jnp.inf, logits)
    v2 = jnp.max(masked, axis=1, keepdims=True)
    i2 = jnp.argmax(masked, axis=1).astype(jnp.int32)[:, None]
    e2 = jnp.exp(v2 - v1)
    g1 = 1.0 / (1.0 + e2)
    g2 = e2 * g1
    gates = jnp.where(iota == i1, g1, 0.0) + jnp.where(iota == i2, g2, 0.0)
    gates_ref[...] = gates
    psum_ref[0, 0, :] = jnp.sum(gates, axis=0)


def _combine_kernel(x_ref, g_ref, w_ref, b_ref, o_ref):
    e = pl.program_id(1)

    @pl.when(e == 0)
    def _():
        o_ref[...] = jnp.zeros_like(o_ref)

    xw = jnp.dot(x_ref[...], w_ref[0],
                 preferred_element_type=jnp.float32) + b_ref[0]
    iota = jax.lax.broadcasted_iota(jnp.int32, g_ref.shape, 1)
    gcol = jnp.sum(jnp.where(iota == e, g_ref[...], 0.0), axis=1,
                   keepdims=True)
    o_ref[...] += gcol * xw


@jax.jit
def kernel(x, w_gate_W, w_gate_b, expert_W, expert_b):
    n_gt = N // _GATE_TILE
    gates, psums = pl.pallas_call(
        _gating_kernel,
        grid=(n_gt,),
        in_specs=[
            pl.BlockSpec((_GATE_TILE, D), lambda i: (i, 0)),
            pl.BlockSpec((D, E), lambda i: (0, 0)),
            pl.BlockSpec((1, E), lambda i: (0, 0)),
        ],
        out_specs=[
            pl.BlockSpec((_GATE_TILE, E), lambda i: (i, 0)),
            pl.BlockSpec((1, 1, E), lambda i: (i, 0, 0)),
        ],
        out_shape=[
            jax.ShapeDtypeStruct((N, E), jnp.float32),
            jax.ShapeDtypeStruct((n_gt, 1, E), jnp.float32),
        ],
        compiler_params=pltpu.CompilerParams(
            dimension_semantics=("parallel",)),
    )(x, w_gate_W, w_gate_b.reshape(1, E))

    n_ct = N // _COMB_TILE
    y = pl.pallas_call(
        _combine_kernel,
        grid=(n_ct, E),
        in_specs=[
            pl.BlockSpec((_COMB_TILE, D), lambda i, e: (i, 0)),
            pl.BlockSpec((_COMB_TILE, E), lambda i, e: (i, 0)),
            pl.BlockSpec((1, D, D), lambda i, e: (e, 0, 0)),
            pl.BlockSpec((1, 1, D), lambda i, e: (e, 0, 0)),
        ],
        out_specs=pl.BlockSpec((_COMB_TILE, D), lambda i, e: (i, 0)),
        out_shape=jax.ShapeDtypeStruct((N, D), jnp.float32),
        compiler_params=pltpu.CompilerParams(
            dimension_semantics=("parallel", "arbitrary")),
    )(x, gates, expert_W, expert_b.reshape(E, 1, D))

    importance = jnp.sum(psums[:, 0, :], axis=0) / N
    loss = (jnp.std(importance, ddof=1) / jnp.mean(importance)) * LOSS_COEF
    return (y, loss, gates)


# speedup vs baseline: 2.5088x; 1.0326x over previous
import jax
import jax.numpy as jnp
from jax.experimental import pallas as pl
from jax.experimental.pallas import tpu as pltpu

N = 8192
D = 1024
E = 8
K = 2
LOSS_COEF = 0.01

_TILE = 2048


def _moe_kernel(x_ref, wg_ref, bg_ref, w_ref, b_ref,
                gates_ref, psum_ref, o_ref, g_scr):
    e = pl.program_id(1)

    @pl.when(e == 0)
    def _():
        logits = jnp.dot(x_ref[...], wg_ref[...],
                         preferred_element_type=jnp.float32) + bg_ref[...]
        iota = jax.lax.broadcasted_iota(jnp.int32, logits.shape, 1)
        v1 = jnp.max(logits, axis=1, keepdims=True)
        i1 = jnp.argmax(logits, axis=1).astype(jnp.int32)[:, None]
        masked = jnp.where(iota == i1, -jnp.inf, logits)
        v2 = jnp.max(masked, axis=1, keepdims=True)
        i2 = jnp.argmax(masked, axis=1).astype(jnp.int32)[:, None]
        e2 = jnp.exp(v2 - v1)
        g1 = 1.0 / (1.0 + e2)
        g2 = e2 * g1
        gates = (jnp.where(iota == i1, g1, 0.0)
                 + jnp.where(iota == i2, g2, 0.0))
        gates_ref[...] = gates
        g_scr[...] = gates
        psum_ref[0, 0, :] = jnp.sum(gates, axis=0)
        o_ref[...] = jnp.zeros_like(o_ref)

    xw = jnp.dot(x_ref[...], w_ref[0],
                 preferred_element_type=jnp.float32) + b_ref[0]
    iota = jax.lax.broadcasted_iota(jnp.int32, g_scr.shape, 1)
    gcol = jnp.sum(jnp.where(iota == e, g_scr[...], 0.0), axis=1,
                   keepdims=True)
    o_ref[...] += gcol * xw


@jax.jit
def kernel(x, w_gate_W, w_gate_b, expert_W, expert_b):
    n_t = N // _TILE
    gates, psums, y = pl.pallas_call(
        _moe_kernel,
        grid=(n_t, E),
        in_specs=[
            pl.BlockSpec((_TILE, D), lambda i, e: (i, 0)),
            pl.BlockSpec((D, E), lambda i, e: (0, 0)),
            pl.BlockSpec((1, E), lambda i, e: (0, 0)),
            pl.BlockSpec((1, D, D), lambda i, e: (e, 0, 0)),
            pl.BlockSpec((1, 1, D), lambda i, e: (e, 0, 0)),
        ],
        out_specs=[
            pl.BlockSpec((_TILE, E), lambda i, e: (i, 0)),
            pl.BlockSpec((1, 1, E), lambda i, e: (i, 0, 0)),
            pl.BlockSpec((_TILE, D), lambda i, e: (i, 0)),
        ],
        out_shape=[
            jax.ShapeDtypeStruct((N, E), jnp.float32),
            jax.ShapeDtypeStruct((n_t, 1, E), jnp.float32),
            jax.ShapeDtypeStruct((N, D), jnp.float32),
        ],
        scratch_shapes=[pltpu.VMEM((_TILE, E), jnp.float32)],
        compiler_params=pltpu.CompilerParams(
            dimension_semantics=("parallel", "arbitrary")),
    )(x, w_gate_W, w_gate_b.reshape(1, E), expert_W,
      expert_b.reshape(E, 1, D))

    importance = jnp.sum(psums[:, 0, :], axis=0) / N
    loss = (jnp.std(importance, ddof=1) / jnp.mean(importance)) * LOSS_COEF
    return (y, loss, gates)
